# baseline (speedup 1.0000x reference)
import functools

import jax
import jax.numpy as jnp
from jax import lax
from jax.experimental import pallas as pl
from jax.experimental.pallas import tpu as pltpu
from jax.experimental.pallas import tpu_sc as plsc

B = 16384
N = 1000
K = 1638
BL = 512
KPAD = 1664
GPAD = 1792
NW = 32
BPW = GPAD // NW

_I32_MAX = 2**31 - 1
_I32_MIN = -2**31


def _entropy_body(x_ref, o_ref):
    x = x_ref[...]
    m = jnp.max(x, axis=0, keepdims=True)
    t = x - m
    e = jnp.exp(t)

    def rsum(a):
        acc = jnp.zeros((8, BL), jnp.float32)
        for v in range(N // 8):
            acc = acc + a[8 * v:8 * v + 8, :]
        h1 = acc[0:4] + acc[4:8]
        h2 = h1[0:2] + h1[2:4]
        return h2[0:1] + h2[1:2]

    s = rsum(e)
    logs = jnp.log(s)
    p = e / s
    q = p * (t - logs)
    o_ref[...] = (-rsum(q)).reshape(1, 1, BL)


def _entropy(xt):
    return pl.pallas_call(
        _entropy_body,
        grid=(B // BL,),
        in_specs=[pl.BlockSpec((N, BL), lambda i: (0, i))],
        out_specs=pl.BlockSpec((1, 1, BL), lambda i: (i, 0, 0)),
        out_shape=jax.ShapeDtypeStruct((B // BL, 1, BL), jnp.float32),
    )(xt)


def _dot(a, b):
    return lax.dot_general(a, b, (((1,), (0,)), ((), ())),
                           precision=lax.Precision.HIGHEST,
                           preferred_element_type=jnp.float32)


def _dot0(a, b):
    return lax.dot_general(a, b, (((0,), (0,)), ((), ())),
                           precision=lax.Precision.HIGHEST,
                           preferred_element_type=jnp.float32)


def _select_body(h_ref, out_ref, comp_ref):
    h = h_ref[...]
    b = lax.bitcast_convert_type(h, jnp.int32)
    key = jnp.where(b < 0, b ^ jnp.int32(_I32_MAX), b)

    def bs(_, lohi):
        lo, hi = lohi
        mid = (lo >> 1) + (hi >> 1) + (lo & hi & 1)
        take = jnp.sum((key <= mid).astype(jnp.int32)) >= K
        return (jnp.where(take, lo, mid + 1), jnp.where(take, mid, hi))

    tau, _ = lax.fori_loop(0, 32, bs, (jnp.int32(_I32_MIN), jnp.int32(_I32_MAX)))

    nless = jnp.sum((key < tau).astype(jnp.int32))
    need = (K - nless).astype(jnp.float32)
    tie = key == tau

    ii = lax.broadcasted_iota(jnp.int32, (128, 128), 0)
    jj = lax.broadcasted_iota(jnp.int32, (128, 128), 1)
    ltri = (ii >= jj).astype(jnp.float32)
    utri = (ii < jj).astype(jnp.float32)

    def cumsum_cm(x):
        y = _dot(ltri, x)
        return y + _dot(y[127:128, :], utri)

    tie_rank = cumsum_cm(tie.astype(jnp.float32))
    sel = (key < tau) | (tie & (tie_rank <= need))
    sel_f = sel.astype(jnp.float32)
    pos = cumsum_cm(sel_f) - sel_f
    rel = pos - pos[0:1, :]

    gidx = (jj * 128 + ii).astype(jnp.float32)
    keyhi = (key >> 16).astype(jnp.float32)
    keylo = (key & 0xFFFF).astype(jnp.float32)

    comp_ref[...] = jnp.zeros((GPAD, 3), jnp.float32)
    wiota = lax.broadcasted_iota(jnp.int32, (1, 128), 1).astype(jnp.float32)
    for c in range(128):
        oh = (rel[:, c:c + 1] == wiota).astype(jnp.float32) * sel_f[:, c:c + 1]
        vals = jnp.concatenate(
            [gidx[:, c:c + 1], keyhi[:, c:c + 1], keylo[:, c:c + 1]], axis=1)
        contrib = _dot0(oh, vals)
        start = pos[0, c].astype(jnp.int32)
        comp_ref[pl.ds(start, 128), :] += contrib

    comp = comp_ref[0:KPAD, :]
    cidx_f = comp[:, 0:1]
    ckey = (comp[:, 1:2].astype(jnp.int32) * 65536
            + comp[:, 2:3].astype(jnp.int32))
    qcol = lax.broadcasted_iota(jnp.int32, (KPAD, 1), 0)
    ckey = jnp.where(qcol >= K, jnp.int32(_I32_MAX), ckey)
    ckey_row = jnp.transpose(ckey)

    lane = lax.broadcasted_iota(jnp.int32, (1, 128), 1)
    ranks = []
    for cc in range(KPAD // 128):
        kchunk = ckey_row[:, 128 * cc:128 * cc + 128]
        cmp = (ckey < kchunk) | ((ckey == kchunk) & (qcol < lane + 128 * cc))
        ranks.append(jnp.sum(cmp.astype(jnp.int32), axis=0, keepdims=True))
    rank_row = jnp.concatenate(ranks, axis=1)

    grow = lax.broadcasted_iota(jnp.int32, (128, 1), 0)
    cols = []
    for gc in range(GPAD // 128):
        oh = (rank_row == grow + 128 * gc).astype(jnp.float32)
        cols.append(_dot(oh, cidx_f))
    out_ref[...] = jnp.concatenate(cols, axis=1).astype(jnp.int32)


def _select(h_cm):
    return pl.pallas_call(
        _select_body,
        out_shape=jax.ShapeDtypeStruct((128, GPAD // 128), jnp.int32),
        scratch_shapes=[pltpu.VMEM((GPAD, 3), jnp.float32)],
    )(h_cm)


NH = 896
NT = 128


@functools.lru_cache(maxsize=1)
def _make_gather():
    mesh = plsc.VectorSubcoreMesh(core_axis_name="c", subcore_axis_name="s")

    @functools.partial(
        pl.kernel,
        mesh=mesh,
        out_type=(
            jax.ShapeDtypeStruct((GPAD, NH), jnp.float32),
            jax.ShapeDtypeStruct((GPAD, NT), jnp.float32),
        ),
        scratch_types=[
            pltpu.VMEM((BPW,), jnp.int32),
            pltpu.VMEM((BPW, NH), jnp.float32),
            pltpu.VMEM((BPW, NT), jnp.float32),
            pltpu.SemaphoreType.DMA,
            pltpu.SemaphoreType.DMA,
        ],
    )
    def gather(table_hbm, tail_hbm, idx_hbm, outh_hbm, outt_hbm,
               idx_v, head_v, tail_v, sem_h, sem_t):
        wid = lax.axis_index("s") * 2 + lax.axis_index("c")
        base = wid * BPW
        pltpu.sync_copy(idx_hbm.at[pl.ds(base, BPW)], idx_v)
        ch = pltpu.async_copy(table_hbm.at[idx_v, pl.ds(0, NH)], head_v, sem_h)
        ct = pltpu.async_copy(tail_hbm.at[idx_v], tail_v, sem_t)
        ch.wait()
        pltpu.sync_copy(head_v, outh_hbm.at[pl.ds(base, BPW)])
        ct.wait()
        pltpu.sync_copy(tail_v, outt_hbm.at[pl.ds(base, BPW)])

    return gather


def kernel(logits):
    xt = jnp.transpose(logits)
    hflat = _entropy(xt).reshape(B)
    h_cm = jnp.transpose(hflat.reshape(128, 128))
    out_cm = _select(h_cm)
    idx_full = jnp.transpose(out_cm).reshape(GPAD)
    tailp = jnp.pad(logits[:, NH:], ((0, 0), (0, NT - (N - NH))))
    head, tail = _make_gather()(logits, tailp, idx_full)
    rows = jnp.concatenate([head, tail[:, :N - NH]], axis=1)
    return rows[:K], idx_full[:K]


# speedup vs baseline: 1.1109x; 1.1109x over previous
import functools

import jax
import jax.numpy as jnp
from jax import lax
from jax.experimental import pallas as pl
from jax.experimental.pallas import tpu as pltpu
from jax.experimental.pallas import tpu_sc as plsc

B = 16384
N = 1000
K = 1638
BL = 512
KPAD = 1792
GPAD = 1792
NW = 32
BPW = GPAD // NW

_I32_MAX = 2**31 - 1
_I32_MIN = -2**31


def _entropy_body(x_ref, o_ref):
    x = x_ref[...]
    m = jnp.max(x, axis=0, keepdims=True)
    t = x - m
    e = jnp.exp(t)

    def rsum(a):
        acc = jnp.zeros((8, BL), jnp.float32)
        for v in range(N // 8):
            acc = acc + a[8 * v:8 * v + 8, :]
        h1 = acc[0:4] + acc[4:8]
        h2 = h1[0:2] + h1[2:4]
        return h2[0:1] + h2[1:2]

    s = rsum(e)
    logs = jnp.log(s)
    p = e * (1.0 / s)
    q = p * (t - logs)
    o_ref[...] = (-rsum(q)).reshape(1, 1, BL)


def _entropy(xt):
    return pl.pallas_call(
        _entropy_body,
        grid=(B // BL,),
        in_specs=[pl.BlockSpec((N, BL), lambda i: (0, i))],
        out_specs=pl.BlockSpec((1, 1, BL), lambda i: (i, 0, 0)),
        out_shape=jax.ShapeDtypeStruct((B // BL, 1, BL), jnp.float32),
    )(xt)


def _dot(a, b):
    return lax.dot_general(a, b, (((1,), (0,)), ((), ())),
                           preferred_element_type=jnp.float32)


def _dot0(a, b):
    return lax.dot_general(a, b, (((0,), (0,)), ((), ())),
                           preferred_element_type=jnp.float32)


def _select_body(h_ref, out_ref, c0_ref, c1_ref, c2_ref, c3_ref):
    h = h_ref[...]
    b = lax.bitcast_convert_type(h, jnp.int32)
    key = jnp.where(b < 0, b ^ jnp.int32(_I32_MAX), b)

    def bs(_, lohi):
        lo, hi = lohi
        mid = (lo >> 1) + (hi >> 1) + (lo & hi & 1)
        take = jnp.sum((key <= mid).astype(jnp.int32)) >= K
        return (jnp.where(take, lo, mid + 1), jnp.where(take, mid, hi))

    tau, _ = lax.fori_loop(0, 32, bs, (jnp.int32(_I32_MIN), jnp.int32(_I32_MAX)))

    nless = jnp.sum((key < tau).astype(jnp.int32))
    need = (K - nless).astype(jnp.float32)
    tie = key == tau

    ii = lax.broadcasted_iota(jnp.int32, (128, 128), 0)
    jj = lax.broadcasted_iota(jnp.int32, (128, 128), 1)
    ltri = (ii >= jj).astype(jnp.float32)
    utri = (ii < jj).astype(jnp.float32)

    def cumsum_cm(x):
        y = _dot(ltri, x)
        return y + _dot(y[127:128, :], utri)

    tie_rank = cumsum_cm(tie.astype(jnp.float32))
    sel = (key < tau) | (tie & (tie_rank <= need))
    sel_f = sel.astype(jnp.float32)
    pos = cumsum_cm(sel_f) - sel_f
    rel = pos - pos[0:1, :]

    ku = lax.bitcast_convert_type(key ^ jnp.int32(_I32_MIN), jnp.uint32)
    vb = [
        ((jj * 128 + ii) >> 7).astype(jnp.float32),
        ((jj * 128 + ii) & 127).astype(jnp.float32),
        (ku >> 24).astype(jnp.float32),
        ((ku >> 16) & 255).astype(jnp.float32),
        ((ku >> 8) & 255).astype(jnp.float32),
        (ku & 255).astype(jnp.float32),
    ]

    rel_t = jnp.transpose(rel)
    sel_t = jnp.transpose(sel_f)
    vb_t = [jnp.transpose(v) for v in vb]

    bufs = [c0_ref, c1_ref, c2_ref, c3_ref]
    for buf in bufs:
        buf[...] = jnp.zeros((KPAD, 6), jnp.float32)
    wiota = lax.broadcasted_iota(jnp.int32, (1, 128), 1)
    wiota_f = wiota.astype(jnp.float32)
    wcol_f = lax.broadcasted_iota(jnp.int32, (128, 1), 0).astype(jnp.float32)
    for c in range(128):
        oh_t = ((rel_t[c:c + 1, :] == wcol_f).astype(jnp.float32)
                * sel_t[c:c + 1, :])
        vals_t = jnp.concatenate([v[c:c + 1, :] for v in vb_t], axis=0)
        contrib = lax.dot_general(oh_t, vals_t, (((1,), (1,)), ((), ())),
                                  preferred_element_type=jnp.float32)
        start = pos[0, c].astype(jnp.int32)
        bufs[c % 4][pl.ds(start, 128), :] += contrib

    comp = c0_ref[...] + c1_ref[...] + c2_ref[...] + c3_ref[...]
    cidx_f = comp[:, 0:1] * 128.0 + comp[:, 1:2]
    hi16 = (comp[:, 2:3] * 256.0 + comp[:, 3:4]).astype(jnp.int32)
    lo16 = (comp[:, 4:5] * 256.0 + comp[:, 5:6]).astype(jnp.int32)
    ckey = (hi16 - 32768) * 65536 + lo16
    qcol = lax.broadcasted_iota(jnp.int32, (KPAD, 1), 0)
    ckey = jnp.where(qcol >= K, jnp.int32(_I32_MAX), ckey)
    cidx_f = jnp.where(qcol >= K, 0.0, cidx_f)
    ckey_row = jnp.transpose(ckey)
    qrow = lax.broadcasted_iota(jnp.int32, (1, KPAD), 1)
    rsub = lax.broadcasted_iota(jnp.int32, (128, 1), 0)

    rank_chunks = []
    for cc in range(KPAD // 128):
        kcol = ckey[128 * cc:128 * cc + 128, :]
        cmp = (ckey_row < kcol) | ((ckey_row == kcol) & (qrow < rsub + 128 * cc))
        rank_chunks.append(
            jnp.sum(cmp.astype(jnp.int32), axis=1, keepdims=True))
    rank_col = jnp.concatenate(rank_chunks, axis=0)

    ci = cidx_f.astype(jnp.int32)
    phiota = lax.broadcasted_iota(jnp.int32, (1, GPAD // 128), 1)
    a_low = (((rank_col & 127) == wiota)).astype(jnp.float32)
    phm = ((rank_col >> 7) == phiota).astype(jnp.float32)
    bmat = jnp.concatenate(
        [phm * (ci >> 7).astype(jnp.float32),
         phm * (ci & 127).astype(jnp.float32)], axis=1)
    out2 = _dot0(a_low, bmat)
    np14 = GPAD // 128
    out_ref[...] = (out2[:, 0:np14] * 128.0
                    + out2[:, np14:2 * np14]).astype(jnp.int32)


def _select(h_cm):
    return pl.pallas_call(
        _select_body,
        out_shape=jax.ShapeDtypeStruct((128, GPAD // 128), jnp.int32),
        scratch_shapes=[pltpu.VMEM((KPAD, 6), jnp.float32) for _ in range(4)],
    )(h_cm)


NH = 896
NT = 128


@functools.lru_cache(maxsize=1)
def _make_gather():
    mesh = plsc.VectorSubcoreMesh(core_axis_name="c", subcore_axis_name="s")

    @functools.partial(
        pl.kernel,
        mesh=mesh,
        out_type=(
            jax.ShapeDtypeStruct((GPAD, NH), jnp.float32),
            jax.ShapeDtypeStruct((GPAD, NT), jnp.float32),
        ),
        scratch_types=[
            pltpu.VMEM((BPW,), jnp.int32),
            pltpu.VMEM((BPW, NH), jnp.float32),
            pltpu.VMEM((BPW, NT), jnp.float32),
            pltpu.SemaphoreType.DMA,
            pltpu.SemaphoreType.DMA,
        ],
    )
    def gather(table_hbm, tail_hbm, idx_hbm, outh_hbm, outt_hbm,
               idx_v, head_v, tail_v, sem_h, sem_t):
        wid = lax.axis_index("s") * 2 + lax.axis_index("c")
        base = wid * BPW
        pltpu.sync_copy(idx_hbm.at[pl.ds(base, BPW)], idx_v)
        ch = pltpu.async_copy(table_hbm.at[idx_v, pl.ds(0, NH)], head_v, sem_h)
        ct = pltpu.async_copy(tail_hbm.at[idx_v], tail_v, sem_t)
        ch.wait()
        pltpu.sync_copy(head_v, outh_hbm.at[pl.ds(base, BPW)])
        ct.wait()
        pltpu.sync_copy(tail_v, outt_hbm.at[pl.ds(base, BPW)])

    return gather


def kernel(logits):
    xt = jnp.transpose(logits)
    hflat = _entropy(xt).reshape(B)
    h_cm = jnp.transpose(hflat.reshape(128, 128))
    out_cm = _select(h_cm)
    idx_full = jnp.transpose(out_cm).reshape(GPAD)
    tailp = jnp.pad(logits[:, NH:], ((0, 0), (0, NT - (N - NH))))
    head, tail = _make_gather()(logits, tailp, idx_full)
    rows = jnp.concatenate([head, tail[:, :N - NH]], axis=1)
    return rows[:K], idx_full[:K]


# speedup vs baseline: 1.1115x; 1.0005x over previous
import functools

import jax
import jax.numpy as jnp
from jax import lax
from jax.experimental import pallas as pl
from jax.experimental.pallas import tpu as pltpu
from jax.experimental.pallas import tpu_sc as plsc

B = 16384
N = 1000
K = 1638
BL = 1024
KPAD = 1792
GPAD = 1792
NW = 32
BPW = GPAD // NW

_I32_MAX = 2**31 - 1
_I32_MIN = -2**31


def _entropy_body(x_ref, o_ref):
    x = jnp.transpose(x_ref[...])
    m = jnp.max(x, axis=0, keepdims=True)
    t = x - m
    e = jnp.exp(t)

    def rsum(a):
        acc = jnp.zeros((8, BL), jnp.float32)
        for v in range(N // 8):
            acc = acc + a[8 * v:8 * v + 8, :]
        h1 = acc[0:4] + acc[4:8]
        h2 = h1[0:2] + h1[2:4]
        return h2[0:1] + h2[1:2]

    s = rsum(e)
    logs = jnp.log(s)
    p = e * (1.0 / s)
    q = p * (t - logs)
    h = -rsum(q)
    o_ref[...] = h.reshape(BL // 128, 128)


def _entropy(logits):
    return pl.pallas_call(
        _entropy_body,
        grid=(B // BL,),
        in_specs=[pl.BlockSpec((BL, N), lambda i: (i, 0))],
        out_specs=pl.BlockSpec((BL // 128, 128), lambda i: (i, 0)),
        out_shape=jax.ShapeDtypeStruct((128, 128), jnp.float32),
    )(logits)


def _dot(a, b):
    return lax.dot_general(a, b, (((1,), (0,)), ((), ())),
                           preferred_element_type=jnp.float32)


def _dot0(a, b):
    return lax.dot_general(a, b, (((0,), (0,)), ((), ())),
                           preferred_element_type=jnp.float32)


def _select_body(h_ref, out_ref, c0_ref, c1_ref, c2_ref, c3_ref):
    h = jnp.transpose(h_ref[...])
    b = lax.bitcast_convert_type(h, jnp.int32)
    key = jnp.where(b < 0, b ^ jnp.int32(_I32_MAX), b)

    def bs(_, lohi):
        lo, hi = lohi
        mid = (lo >> 1) + (hi >> 1) + (lo & hi & 1)
        take = jnp.sum((key <= mid).astype(jnp.int32)) >= K
        return (jnp.where(take, lo, mid + 1), jnp.where(take, mid, hi))

    tau, _ = lax.fori_loop(0, 32, bs, (jnp.int32(_I32_MIN), jnp.int32(_I32_MAX)))

    nless = jnp.sum((key < tau).astype(jnp.int32))
    need = (K - nless).astype(jnp.float32)
    tie = key == tau

    ii = lax.broadcasted_iota(jnp.int32, (128, 128), 0)
    jj = lax.broadcasted_iota(jnp.int32, (128, 128), 1)
    ltri = (ii >= jj).astype(jnp.float32)
    utri = (ii < jj).astype(jnp.float32)

    def cumsum_cm(x):
        y = _dot(ltri, x)
        return y + _dot(y[127:128, :], utri)

    tie_rank = cumsum_cm(tie.astype(jnp.float32))
    sel = (key < tau) | (tie & (tie_rank <= need))
    sel_f = sel.astype(jnp.float32)
    pos = cumsum_cm(sel_f) - sel_f
    rel = pos - pos[0:1, :]

    ku = lax.bitcast_convert_type(key ^ jnp.int32(_I32_MIN), jnp.uint32)
    vb = [
        ((jj * 128 + ii) >> 7).astype(jnp.float32),
        ((jj * 128 + ii) & 127).astype(jnp.float32),
        (ku >> 24).astype(jnp.float32),
        ((ku >> 16) & 255).astype(jnp.float32),
        ((ku >> 8) & 255).astype(jnp.float32),
        (ku & 255).astype(jnp.float32),
    ]

    rel_t = jnp.transpose(rel)
    sel_t = jnp.transpose(sel_f)
    vb_t = [jnp.transpose(v) for v in vb]

    bufs = [c0_ref, c1_ref, c2_ref, c3_ref]
    for buf in bufs:
        buf[...] = jnp.zeros((KPAD, 6), jnp.float32)
    wiota = lax.broadcasted_iota(jnp.int32, (1, 128), 1)
    wiota_f = wiota.astype(jnp.float32)
    wcol_f = lax.broadcasted_iota(jnp.int32, (128, 1), 0).astype(jnp.float32)
    for c in range(128):
        oh_t = ((rel_t[c:c + 1, :] == wcol_f).astype(jnp.float32)
                * sel_t[c:c + 1, :])
        vals_t = jnp.concatenate([v[c:c + 1, :] for v in vb_t], axis=0)
        contrib = lax.dot_general(oh_t, vals_t, (((1,), (1,)), ((), ())),
                                  preferred_element_type=jnp.float32)
        start = pos[0, c].astype(jnp.int32)
        bufs[c % 4][pl.ds(start, 128), :] += contrib

    comp = c0_ref[...] + c1_ref[...] + c2_ref[...] + c3_ref[...]
    cidx_f = comp[:, 0:1] * 128.0 + comp[:, 1:2]
    hi16 = (comp[:, 2:3] * 256.0 + comp[:, 3:4]).astype(jnp.int32)
    lo16 = (comp[:, 4:5] * 256.0 + comp[:, 5:6]).astype(jnp.int32)
    ckey = (hi16 - 32768) * 65536 + lo16
    qcol = lax.broadcasted_iota(jnp.int32, (KPAD, 1), 0)
    ckey = jnp.where(qcol >= K, jnp.int32(_I32_MAX), ckey)
    cidx_f = jnp.where(qcol >= K, 0.0, cidx_f)
    ckey_row = jnp.transpose(ckey)
    qrow = lax.broadcasted_iota(jnp.int32, (1, KPAD), 1)
    rsub = lax.broadcasted_iota(jnp.int32, (128, 1), 0)

    rank_chunks = []
    for cc in range(KPAD // 128):
        kcol = ckey[128 * cc:128 * cc + 128, :]
        cmp = (ckey_row < kcol) | ((ckey_row == kcol) & (qrow < rsub + 128 * cc))
        rank_chunks.append(
            jnp.sum(cmp.astype(jnp.int32), axis=1, keepdims=True))
    rank_col = jnp.concatenate(rank_chunks, axis=0)

    ci = cidx_f.astype(jnp.int32)
    phiota = lax.broadcasted_iota(jnp.int32, (1, GPAD // 128), 1)
    a_low = (((rank_col & 127) == wiota)).astype(jnp.float32)
    phm = ((rank_col >> 7) == phiota).astype(jnp.float32)
    bmat = jnp.concatenate(
        [phm * (ci >> 7).astype(jnp.float32),
         phm * (ci & 127).astype(jnp.float32)], axis=1)
    out2 = _dot0(a_low, bmat)
    np14 = GPAD // 128
    out_ref[...] = (out2[:, 0:np14] * 128.0
                    + out2[:, np14:2 * np14]).astype(jnp.int32)


def _select(h_cm):
    return pl.pallas_call(
        _select_body,
        out_shape=jax.ShapeDtypeStruct((128, GPAD // 128), jnp.int32),
        scratch_shapes=[pltpu.VMEM((KPAD, 6), jnp.float32) for _ in range(4)],
    )(h_cm)


NH = 896
NT = 128


@functools.lru_cache(maxsize=1)
def _make_gather():
    mesh = plsc.VectorSubcoreMesh(core_axis_name="c", subcore_axis_name="s")

    @functools.partial(
        pl.kernel,
        mesh=mesh,
        out_type=(
            jax.ShapeDtypeStruct((GPAD, NH), jnp.float32),
            jax.ShapeDtypeStruct((GPAD, NT), jnp.float32),
        ),
        scratch_types=[
            pltpu.VMEM((BPW,), jnp.int32),
            pltpu.VMEM((BPW, NH), jnp.float32),
            pltpu.VMEM((BPW, NT), jnp.float32),
            pltpu.SemaphoreType.DMA,
            pltpu.SemaphoreType.DMA,
        ],
    )
    def gather(table_hbm, tail_hbm, idx_hbm, outh_hbm, outt_hbm,
               idx_v, head_v, tail_v, sem_h, sem_t):
        wid = lax.axis_index("s") * 2 + lax.axis_index("c")
        base = wid * BPW
        pltpu.sync_copy(idx_hbm.at[pl.ds(base, BPW)], idx_v)
        ch = pltpu.async_copy(table_hbm.at[idx_v, pl.ds(0, NH)], head_v, sem_h)
        ct = pltpu.async_copy(tail_hbm.at[idx_v], tail_v, sem_t)
        ch.wait()
        pltpu.sync_copy(head_v, outh_hbm.at[pl.ds(base, BPW)])
        ct.wait()
        pltpu.sync_copy(tail_v, outt_hbm.at[pl.ds(base, BPW)])

    return gather


def kernel(logits):
    h_cm_t = _entropy(logits)
    out_cm = _select(h_cm_t)
    idx_full = jnp.transpose(out_cm).reshape(GPAD)
    tailp = jnp.pad(logits[:, NH:], ((0, 0), (0, NT - (N - NH))))
    head, tail = _make_gather()(logits, tailp, idx_full)
    rows = jnp.concatenate([head, tail[:, :N - NH]], axis=1)
    return rows[:K], idx_full[:K]


# speedup vs baseline: 1.1462x; 1.0313x over previous
import functools

import jax
import jax.numpy as jnp
from jax import lax
from jax.experimental import pallas as pl
from jax.experimental.pallas import tpu as pltpu
from jax.experimental.pallas import tpu_sc as plsc

B = 16384
N = 1000
K = 1638
BL = 1024
KPAD = 1792
GPAD = 1792
NW = 32
BPW = GPAD // NW

_I32_MAX = 2**31 - 1
_I32_MIN = -2**31


def _entropy_body(x_ref, o_ref):
    x = x_ref[...]
    m = jnp.max(x, axis=0, keepdims=True)
    t = x - m
    e = jnp.exp(t)

    def rsum(a):
        acc = jnp.zeros((8, BL), jnp.float32)
        for v in range(N // 8):
            acc = acc + a[8 * v:8 * v + 8, :]
        h1 = acc[0:4] + acc[4:8]
        h2 = h1[0:2] + h1[2:4]
        return h2[0:1] + h2[1:2]

    s = rsum(e)
    logs = jnp.log(s)
    p = e * (1.0 / s)
    q = p * (t - logs)
    h = -rsum(q)
    o_ref[...] = h.reshape(BL // 128, 128)


def _entropy(xt):
    return pl.pallas_call(
        _entropy_body,
        grid=(B // BL,),
        in_specs=[pl.BlockSpec((N, BL), lambda i: (0, i))],
        out_specs=pl.BlockSpec((BL // 128, 128), lambda i: (i, 0)),
        out_shape=jax.ShapeDtypeStruct((128, 128), jnp.float32),
    )(xt)


def _dot(a, b):
    return lax.dot_general(a, b, (((1,), (0,)), ((), ())),
                           preferred_element_type=jnp.float32)


def _dot0(a, b):
    return lax.dot_general(a, b, (((0,), (0,)), ((), ())),
                           preferred_element_type=jnp.float32)


def _select_body(h_ref, out_ref, c0_ref, c1_ref, c2_ref, c3_ref):
    h = jnp.transpose(h_ref[...])
    b = lax.bitcast_convert_type(h, jnp.int32)
    key = jnp.where(b < 0, b ^ jnp.int32(_I32_MAX), b)

    def bs(_, lohi):
        lo, hi = lohi
        mid = (lo >> 1) + (hi >> 1) + (lo & hi & 1)
        take = jnp.sum((key <= mid).astype(jnp.int32)) >= K
        return (jnp.where(take, lo, mid + 1), jnp.where(take, mid, hi))

    tau, _ = lax.fori_loop(0, 32, bs, (jnp.int32(_I32_MIN), jnp.int32(_I32_MAX)))

    nless = jnp.sum((key < tau).astype(jnp.int32))
    need = (K - nless).astype(jnp.float32)
    tie = key == tau

    ii = lax.broadcasted_iota(jnp.int32, (128, 128), 0)
    jj = lax.broadcasted_iota(jnp.int32, (128, 128), 1)
    ltri = (ii >= jj).astype(jnp.float32)
    utri = (ii < jj).astype(jnp.float32)

    def cumsum_cm(x):
        y = _dot(ltri, x)
        return y + _dot(y[127:128, :], utri)

    tie_rank = cumsum_cm(tie.astype(jnp.float32))
    sel = (key < tau) | (tie & (tie_rank <= need))
    sel_f = sel.astype(jnp.float32)
    pos = cumsum_cm(sel_f) - sel_f
    rel = pos - pos[0:1, :]

    ku = lax.bitcast_convert_type(key ^ jnp.int32(_I32_MIN), jnp.uint32)
    vb = [
        ((jj * 128 + ii) >> 7).astype(jnp.float32),
        ((jj * 128 + ii) & 127).astype(jnp.float32),
        (ku >> 24).astype(jnp.float32),
        ((ku >> 16) & 255).astype(jnp.float32),
        ((ku >> 8) & 255).astype(jnp.float32),
        (ku & 255).astype(jnp.float32),
    ]

    rel_t = jnp.transpose(rel)
    sel_t = jnp.transpose(sel_f)
    vb_t = [jnp.transpose(v) for v in vb]

    bufs = [c0_ref, c1_ref, c2_ref, c3_ref]
    for buf in bufs:
        buf[...] = jnp.zeros((KPAD, 6), jnp.float32)
    wiota = lax.broadcasted_iota(jnp.int32, (1, 128), 1)
    wiota_f = wiota.astype(jnp.float32)
    wcol_f = lax.broadcasted_iota(jnp.int32, (128, 1), 0).astype(jnp.float32)
    for c in range(128):
        oh_t = ((rel_t[c:c + 1, :] == wcol_f).astype(jnp.float32)
                * sel_t[c:c + 1, :])
        vals_t = jnp.concatenate([v[c:c + 1, :] for v in vb_t], axis=0)
        contrib = lax.dot_general(oh_t, vals_t, (((1,), (1,)), ((), ())),
                                  preferred_element_type=jnp.float32)
        start = pos[0, c].astype(jnp.int32)
        bufs[c % 4][pl.ds(start, 128), :] += contrib

    comp = c0_ref[...] + c1_ref[...] + c2_ref[...] + c3_ref[...]
    cidx_f = comp[:, 0:1] * 128.0 + comp[:, 1:2]
    hi16 = (comp[:, 2:3] * 256.0 + comp[:, 3:4]).astype(jnp.int32)
    lo16 = (comp[:, 4:5] * 256.0 + comp[:, 5:6]).astype(jnp.int32)
    ckey = (hi16 - 32768) * 65536 + lo16
    qcol = lax.broadcasted_iota(jnp.int32, (KPAD, 1), 0)
    ckey = jnp.where(qcol >= K, jnp.int32(_I32_MAX), ckey)
    cidx_f = jnp.where(qcol >= K, 0.0, cidx_f)
    ckey_row = jnp.transpose(ckey)
    qrow = lax.broadcasted_iota(jnp.int32, (1, KPAD), 1)
    rsub = lax.broadcasted_iota(jnp.int32, (128, 1), 0)

    rank_chunks = []
    for cc in range(KPAD // 128):
        kcol = ckey[128 * cc:128 * cc + 128, :]
        cmp = (ckey_row < kcol) | ((ckey_row == kcol) & (qrow < rsub + 128 * cc))
        rank_chunks.append(
            jnp.sum(cmp.astype(jnp.int32), axis=1, keepdims=True))
    rank_col = jnp.concatenate(rank_chunks, axis=0)

    ci = cidx_f.astype(jnp.int32)
    phiota = lax.broadcasted_iota(jnp.int32, (1, GPAD // 128), 1)
    a_low = (((rank_col & 127) == wiota)).astype(jnp.float32)
    phm = ((rank_col >> 7) == phiota).astype(jnp.float32)
    bmat = jnp.concatenate(
        [phm * (ci >> 7).astype(jnp.float32),
         phm * (ci & 127).astype(jnp.float32)], axis=1)
    out2 = _dot0(a_low, bmat)
    np14 = GPAD // 128
    out_ref[...] = (out2[:, 0:np14] * 128.0
                    + out2[:, np14:2 * np14]).astype(jnp.int32)


def _select(h_cm):
    return pl.pallas_call(
        _select_body,
        out_shape=jax.ShapeDtypeStruct((128, GPAD // 128), jnp.int32),
        scratch_shapes=[pltpu.VMEM((KPAD, 6), jnp.float32) for _ in range(4)],
    )(h_cm)


NH = 896
NT = 128


@functools.lru_cache(maxsize=1)
def _make_gather():
    mesh = plsc.VectorSubcoreMesh(core_axis_name="c", subcore_axis_name="s")

    @functools.partial(
        pl.kernel,
        mesh=mesh,
        out_type=(
            jax.ShapeDtypeStruct((GPAD, NH), jnp.float32),
            jax.ShapeDtypeStruct((GPAD, NT), jnp.float32),
        ),
        scratch_types=[
            pltpu.VMEM((BPW,), jnp.int32),
            pltpu.VMEM((BPW, NH), jnp.float32),
            pltpu.VMEM((BPW, NT), jnp.float32),
            pltpu.SemaphoreType.DMA,
            pltpu.SemaphoreType.DMA,
        ],
    )
    def gather(table_hbm, tail_hbm, idx_hbm, outh_hbm, outt_hbm,
               idx_v, head_v, tail_v, sem_h, sem_t):
        wid = lax.axis_index("s") * 2 + lax.axis_index("c")
        base = wid * BPW
        pltpu.sync_copy(idx_hbm.at[pl.ds(base, BPW)], idx_v)
        ch = pltpu.async_copy(table_hbm.at[idx_v, pl.ds(0, NH)], head_v, sem_h)
        ct = pltpu.async_copy(tail_hbm.at[idx_v], tail_v, sem_t)
        ch.wait()
        pltpu.sync_copy(head_v, outh_hbm.at[pl.ds(base, BPW)])
        ct.wait()
        pltpu.sync_copy(tail_v, outt_hbm.at[pl.ds(base, BPW)])

    return gather


def kernel(logits):
    h_cm_t = _entropy(jnp.transpose(logits))
    out_cm = _select(h_cm_t)
    idx_full = jnp.transpose(out_cm).reshape(GPAD)
    tailp = jnp.pad(logits[:, NH:], ((0, 0), (0, NT - (N - NH))))
    head, tail = _make_gather()(logits, tailp, idx_full)
    rows = jnp.concatenate([head, tail[:, :N - NH]], axis=1)
    return rows[:K], idx_full[:K]


# speedup vs baseline: 1.1475x; 1.0011x over previous
import functools

import jax
import jax.numpy as jnp
from jax import lax
from jax.experimental import pallas as pl
from jax.experimental.pallas import tpu as pltpu
from jax.experimental.pallas import tpu_sc as plsc

B = 16384
N = 1000
K = 1638
BL = 1024
KPAD = 1792
GPAD = 1792
NW = 32
BPW = GPAD // NW

_I32_MAX = 2**31 - 1
_I32_MIN = -2**31


def _entropy_body(x_ref, o_ref):
    x = x_ref[...]
    m = jnp.max(x, axis=0, keepdims=True)
    t = x - m
    e = jnp.exp(t)

    def rsum(a):
        acc = jnp.zeros((8, BL), jnp.float32)
        for v in range(N // 8):
            acc = acc + a[8 * v:8 * v + 8, :]
        h1 = acc[0:4] + acc[4:8]
        h2 = h1[0:2] + h1[2:4]
        return h2[0:1] + h2[1:2]

    s = rsum(e)
    logs = jnp.log(s)
    p = e * (1.0 / s)
    q = p * (t - logs)
    h = -rsum(q)
    o_ref[...] = h.reshape(BL // 128, 128)


def _entropy(xt):
    return pl.pallas_call(
        _entropy_body,
        grid=(B // BL,),
        in_specs=[pl.BlockSpec((N, BL), lambda i: (0, i))],
        out_specs=pl.BlockSpec((BL // 128, 128), lambda i: (i, 0)),
        out_shape=jax.ShapeDtypeStruct((128, 128), jnp.float32),
    )(xt)


def _dot(a, b):
    return lax.dot_general(a, b, (((1,), (0,)), ((), ())),
                           preferred_element_type=jnp.float32)


def _dot0(a, b):
    return lax.dot_general(a, b, (((0,), (0,)), ((), ())),
                           preferred_element_type=jnp.float32)


def _select_body(h_ref, out_ref, c0_ref, c1_ref, c2_ref, c3_ref):
    h = jnp.transpose(h_ref[...])
    b = lax.bitcast_convert_type(h, jnp.int32)
    key = jnp.where(b < 0, b ^ jnp.int32(_I32_MAX), b)

    def bs(_, lohi):
        lo, hi = lohi
        mid = (lo >> 1) + (hi >> 1) + (lo & hi & 1)
        take = jnp.sum((key <= mid).astype(jnp.int32)) >= K
        return (jnp.where(take, lo, mid + 1), jnp.where(take, mid, hi))

    tau, _ = lax.fori_loop(0, 32, bs, (jnp.int32(_I32_MIN), jnp.int32(_I32_MAX)))

    nless = jnp.sum((key < tau).astype(jnp.int32))
    need = (K - nless).astype(jnp.float32)
    tie = key == tau

    ii = lax.broadcasted_iota(jnp.int32, (128, 128), 0)
    jj = lax.broadcasted_iota(jnp.int32, (128, 128), 1)
    ltri = (ii >= jj).astype(jnp.float32)
    utri = (ii < jj).astype(jnp.float32)

    def cumsum_cm(x):
        y = _dot(ltri, x)
        return y + _dot(y[127:128, :], utri)

    tie_rank = cumsum_cm(tie.astype(jnp.float32))
    sel = (key < tau) | (tie & (tie_rank <= need))
    sel_f = sel.astype(jnp.float32)
    pos = cumsum_cm(sel_f) - sel_f
    rel = pos - pos[0:1, :]

    ku = lax.bitcast_convert_type(key ^ jnp.int32(_I32_MIN), jnp.uint32)
    vb = [
        ((jj * 128 + ii) >> 7).astype(jnp.float32),
        ((jj * 128 + ii) & 127).astype(jnp.float32),
        (ku >> 24).astype(jnp.float32),
        ((ku >> 16) & 255).astype(jnp.float32),
        ((ku >> 8) & 255).astype(jnp.float32),
        (ku & 255).astype(jnp.float32),
    ]

    rel_t = jnp.transpose(rel)
    sel_t = jnp.transpose(sel_f)
    vb_t = [jnp.transpose(v) for v in vb]

    bufs = [c0_ref, c1_ref, c2_ref, c3_ref]
    for buf in bufs:
        buf[...] = jnp.zeros((KPAD, 6), jnp.float32)
    wiota = lax.broadcasted_iota(jnp.int32, (1, 128), 1)
    wiota_f = wiota.astype(jnp.float32)
    wcol_f = lax.broadcasted_iota(jnp.int32, (128, 1), 0).astype(jnp.float32)
    for c in range(128):
        oh_t = ((rel_t[c:c + 1, :] == wcol_f).astype(jnp.float32)
                * sel_t[c:c + 1, :])
        vals_t = jnp.concatenate([v[c:c + 1, :] for v in vb_t], axis=0)
        contrib = lax.dot_general(oh_t, vals_t, (((1,), (1,)), ((), ())),
                                  preferred_element_type=jnp.float32)
        start = pos[0, c].astype(jnp.int32)
        bufs[c % 4][pl.ds(start, 128), :] += contrib

    comp = c0_ref[...] + c1_ref[...] + c2_ref[...] + c3_ref[...]
    cidx_f = comp[:, 0:1] * 128.0 + comp[:, 1:2]
    hi16 = (comp[:, 2:3] * 256.0 + comp[:, 3:4]).astype(jnp.int32)
    lo16 = (comp[:, 4:5] * 256.0 + comp[:, 5:6]).astype(jnp.int32)
    ckey = (hi16 - 32768) * 65536 + lo16
    qcol = lax.broadcasted_iota(jnp.int32, (KPAD, 1), 0)
    ckey = jnp.where(qcol >= K, jnp.int32(_I32_MAX), ckey)
    cidx_f = jnp.where(qcol >= K, 0.0, cidx_f)
    ckey_row = jnp.transpose(ckey)
    qrow = lax.broadcasted_iota(jnp.int32, (1, KPAD), 1)
    rsub = lax.broadcasted_iota(jnp.int32, (128, 1), 0)

    rank_chunks = []
    for cc in range(KPAD // 128):
        kcol = ckey[128 * cc:128 * cc + 128, :]
        cmp = (ckey_row < kcol) | ((ckey_row == kcol) & (qrow < rsub + 128 * cc))
        rank_chunks.append(
            jnp.sum(cmp.astype(jnp.int32), axis=1, keepdims=True))
    rank_col = jnp.concatenate(rank_chunks, axis=0)

    ci = cidx_f.astype(jnp.int32)
    phiota = lax.broadcasted_iota(jnp.int32, (1, GPAD // 128), 1)
    a_low = (((rank_col & 127) == wiota)).astype(jnp.float32)
    phm = ((rank_col >> 7) == phiota).astype(jnp.float32)
    bmat = jnp.concatenate(
        [phm * (ci >> 7).astype(jnp.float32),
         phm * (ci & 127).astype(jnp.float32)], axis=1)
    out2 = _dot0(a_low, bmat)
    np14 = GPAD // 128
    out_cm = out2[:, 0:np14] * 128.0 + out2[:, np14:2 * np14]
    out_ref[...] = jnp.transpose(out_cm).astype(jnp.int32)


def _select(h_cm):
    return pl.pallas_call(
        _select_body,
        out_shape=jax.ShapeDtypeStruct((GPAD // 128, 128), jnp.int32),
        scratch_shapes=[pltpu.VMEM((KPAD, 6), jnp.float32) for _ in range(4)],
    )(h_cm)


NH = 896
NT = 128


@functools.lru_cache(maxsize=1)
def _make_gather():
    mesh = plsc.VectorSubcoreMesh(core_axis_name="c", subcore_axis_name="s")

    @functools.partial(
        pl.kernel,
        mesh=mesh,
        out_type=jax.ShapeDtypeStruct((GPAD, NH + NT), jnp.float32),
        scratch_types=[
            pltpu.VMEM((BPW,), jnp.int32),
            pltpu.VMEM((BPW, NH), jnp.float32),
            pltpu.VMEM((BPW, NT), jnp.float32),
            pltpu.SemaphoreType.DMA,
            pltpu.SemaphoreType.DMA,
        ],
    )
    def gather(table_hbm, tail_hbm, idx_hbm, out_hbm,
               idx_v, head_v, tail_v, sem_h, sem_t):
        wid = lax.axis_index("s") * 2 + lax.axis_index("c")
        base = wid * BPW
        pltpu.sync_copy(idx_hbm.at[pl.ds(base, BPW)], idx_v)
        ch = pltpu.async_copy(table_hbm.at[idx_v, pl.ds(0, NH)], head_v, sem_h)
        ct = pltpu.async_copy(tail_hbm.at[idx_v], tail_v, sem_t)
        ch.wait()
        pltpu.sync_copy(head_v, out_hbm.at[pl.ds(base, BPW), pl.ds(0, NH)])
        ct.wait()
        pltpu.sync_copy(tail_v, out_hbm.at[pl.ds(base, BPW), pl.ds(NH, NT)])

    return gather


def kernel(logits):
    h_cm_t = _entropy(jnp.transpose(logits))
    idx_full = _select(h_cm_t).reshape(GPAD)
    tailp = jnp.pad(logits[:, NH:], ((0, 0), (0, NT - (N - NH))))
    out = _make_gather()(logits, tailp, idx_full)
    return out[:K, :N], idx_full[:K]
